# baseline (speedup 1.0000x reference)
import functools
import math as _math

import jax
import jax.numpy as jnp
from jax import lax
from jax.experimental import pallas as pl
from jax.experimental.pallas import tpu as pltpu
from jax.experimental.pallas import tpu_sc as plsc

_V = 32000
_N = 4096
_SMOOTHING = 0.1
_CONF = 1.0 - _SMOOTHING
_EPS = _SMOOTHING / (_V - 2)
_C = _SMOOTHING * _math.log(_EPS) + _CONF * _math.log(_CONF)

_NT = 2048
_R = 128

_NC = 2
_NS = 16
_NW = _NC * _NS
_LANES = 16
_TCG = _NT // _NW
_RPW = (_N - _NT) // _NW
_VREGS = _V // _LANES
_UNROLL = 25
_OUTER = _VREGS // _UNROLL


def _tc_body(x_ref, t_ref, o_ref):
    i = pl.program_id(0)
    s = jnp.sum(x_ref[...], axis=1) - x_ref[:, 0]
    tot = jnp.sum(jnp.where(t_ref[...] != 0, s, 0.0))

    @pl.when(i == 0)
    def _():
        o_ref[0, 0] = tot

    @pl.when(i > 0)
    def _():
        o_ref[0, 0] = o_ref[0, 0] + tot


def _tc_masked_adj_sum(log_probs, target):
    return pl.pallas_call(
        _tc_body,
        grid=(_NT // _R,),
        in_specs=[
            pl.BlockSpec((_R, _V), lambda i: (i, 0)),
            pl.BlockSpec((_R,), lambda i: (i,)),
        ],
        out_specs=pl.BlockSpec(memory_space=pltpu.SMEM),
        out_shape=jax.ShapeDtypeStruct((1, 1), jnp.float32),
        compiler_params=pltpu.CompilerParams(
            dimension_semantics=("arbitrary",)
        ),
    )(log_probs, target)


def _row_sum_vec(buf_ref):
    def body(j, acc):
        off = j * (_UNROLL * _LANES)
        for k in range(_UNROLL):
            acc = acc + buf_ref[pl.ds(off + k * _LANES, _LANES)]
        return acc

    return lax.fori_loop(0, _OUTER, body, jnp.zeros((_LANES,), jnp.float32))


def _take16(v, idx):
    return lax.gather(
        v, idx[:, None],
        dimension_numbers=lax.GatherDimensionNumbers(
            offset_dims=(), collapsed_slice_dims=(0,), start_index_map=(0,)
        ),
        slice_sizes=(1,),
        mode=lax.GatherScatterMode.PROMISE_IN_BOUNDS,
    )


def _allsum_bc(v, lane):
    for sh in (8, 4, 2, 1):
        v = v + _take16(v, lane ^ sh)
    return v


@functools.cache
def _build_sc_mix():
    mesh = plsc.VectorSubcoreMesh(
        core_axis_name="c", subcore_axis_name="s", num_cores=_NC
    )

    @functools.partial(
        pl.kernel,
        mesh=mesh,
        out_type=jax.ShapeDtypeStruct((_NW, _LANES), jnp.float32),
        scratch_types=[
            pltpu.VMEM((_TCG + _RPW,), jnp.int32),
            pltpu.VMEM((_TCG + _RPW,), jnp.int32),
            pltpu.VMEM((_TCG + _RPW,), jnp.float32),
            pltpu.VMEM((_V,), jnp.float32),
            pltpu.VMEM((_V,), jnp.float32),
            pltpu.VMEM((_LANES,), jnp.float32),
            pltpu.SemaphoreType.DMA,
            pltpu.SemaphoreType.DMA,
            pltpu.SemaphoreType.DMA,
        ],
    )
    def _sc_mix(lp2d_hbm, lp_flat_hbm, tgt_hbm, out_hbm,
                tgt_v, idx_v, gat_v, buf0, buf1, acc_v,
                sem_g, sem0, sem1):
        wid = lax.axis_index("s") * _NC + lax.axis_index("c")
        tbase = wid * _TCG
        sbase = _NT + wid * _RPW

        pltpu.async_copy(lp2d_hbm.at[sbase], buf0, sem0)
        pltpu.async_copy(lp2d_hbm.at[sbase + 1], buf1, sem1)

        pltpu.sync_copy(tgt_hbm.at[pl.ds(tbase, _TCG)],
                        tgt_v.at[pl.ds(0, _TCG)])
        pltpu.sync_copy(tgt_hbm.at[pl.ds(sbase, _RPW)],
                        tgt_v.at[pl.ds(_TCG, _RPW)])

        lane = lax.iota(jnp.int32, _LANES)
        for c in range(_TCG // _LANES):
            t16 = tgt_v[pl.ds(c * _LANES, _LANES)]
            row = (tbase + c * _LANES) + lane
            idx_v[pl.ds(c * _LANES, _LANES)] = row * _V + t16
        for c in range(_RPW // _LANES):
            o = _TCG + c * _LANES
            t16 = tgt_v[pl.ds(o, _LANES)]
            row = (sbase + c * _LANES) + lane
            idx_v[pl.ds(o, _LANES)] = row * _V + t16

        pltpu.async_copy(lp_flat_hbm.at[idx_v], gat_v, sem_g).wait()

        acc = jnp.zeros((_LANES,), jnp.float32)
        for c in range(_TCG // _LANES):
            t16 = tgt_v[pl.ds(c * _LANES, _LANES)]
            g16 = gat_v[pl.ds(c * _LANES, _LANES)]
            contrib = _C - (_CONF - _EPS) * g16
            acc = acc + jnp.where(t16 != 0, contrib, jnp.float32(0.0))

        zero16 = jnp.zeros((_LANES,), jnp.float32)

        def group_body(cg, acc):
            grp = zero16
            for l in range(_LANES):
                buf, sem = (buf0, sem0) if l % 2 == 0 else (buf1, sem1)
                pltpu.make_async_copy(lp2d_hbm.at[sbase], buf, sem).wait()
                rsb = _allsum_bc(_row_sum_vec(buf), lane)
                lp0b = _take16(buf[pl.ds(0, _LANES)], lane & 0)
                nxt = cg * _LANES + l + 2

                @pl.when(nxt < _RPW)
                def _():
                    pltpu.async_copy(lp2d_hbm.at[sbase + nxt], buf, sem)

                grp = jnp.where(lane == l, rsb - lp0b, grp)

            o = _TCG + cg * _LANES
            t16 = tgt_v[pl.ds(o, _LANES)]
            g16 = gat_v[pl.ds(o, _LANES)]
            c16 = _C - _EPS * grp - (_CONF - _EPS) * g16
            return acc + jnp.where(t16 != 0, c16, jnp.float32(0.0))

        acc = lax.fori_loop(0, _RPW // _LANES, group_body, acc)
        acc_v[...] = acc
        pltpu.sync_copy(acc_v, out_hbm.at[wid])

    return _sc_mix


def kernel(log_probs, target):
    tgt = target.astype(jnp.int32)
    adj_tc = _tc_masked_adj_sum(log_probs, tgt)
    partials = _build_sc_mix()(log_probs, log_probs.reshape(-1), tgt)
    return jnp.sum(partials) - jnp.float32(_EPS) * adj_tc[0, 0]
